# baseline (speedup 1.0000x reference)
import jax
import jax.numpy as jnp
from jax import lax
from jax.experimental import pallas as pl
from jax.experimental.pallas import tpu as pltpu
from jax.experimental.pallas import tpu_sc as plsc

N_EDGES = 320000
STAGE_E = 1024
N_BLOCKS = 313
E_PAD = N_BLOCKS * STAGE_E
N_NODES_P = 10240
DUMP_ROW = N_NODES_P - 1
E_PER_TILE = 10240
N_STAGES = E_PER_TILE // STAGE_E
CHUNK = 128
CHUNKS_PER_STAGE = STAGE_E // CHUNK
N_CHUNKS = E_PER_TILE // CHUNK
NC = 2
NS = 16
NW = NC * NS
ROWS_PER_TILE = N_NODES_P // NS
TOTAL_STAGES = E_PAD // STAGE_E


def _pack_body(ea_ref, ei_ref, eap_ref, idxp_ref):
    blk = pl.program_id(0)
    t = ea_ref[...].T
    eap_ref[...] = jnp.concatenate(
        [t[j * 128:(j + 1) * 128, :] for j in range(8)], axis=1)
    s = ei_ref[0:1, :].reshape(8, 128)
    r = lax.broadcasted_iota(jnp.int32, (8, 128), 0)
    c = lax.broadcasted_iota(jnp.int32, (8, 128), 1)
    slot = blk * STAGE_E + r * 128 + c
    idxp_ref[...] = jnp.where(slot < N_EDGES, s, DUMP_ROW)


def _pack(ea_t, edge_index):
    return pl.pallas_call(
        _pack_body,
        grid=(N_BLOCKS,),
        in_specs=[
            pl.BlockSpec((16, STAGE_E), lambda i: (0, i)),
            pl.BlockSpec((2, STAGE_E), lambda i: (0, i)),
        ],
        out_specs=[
            pl.BlockSpec((128, 128), lambda i: (i, 0)),
            pl.BlockSpec((8, 128), lambda i: (i, 0)),
        ],
        out_shape=[
            jax.ShapeDtypeStruct((E_PAD // 8, 128), jnp.float32),
            jax.ShapeDtypeStruct((E_PAD // 128, 128), jnp.int32),
        ],
    )(ea_t, edge_index)


def _sc_scatter_body(src_hbm, ea_hbm, sums_out, cnts_out,
                     raw_buf, idx_buf, ea_buf0, ea_buf1, ea_buf2, ones_buf,
                     zb, acc, cnt, sem0, sem1, sem2, ssem):
    c = lax.axis_index("c")
    s = lax.axis_index("s")
    w = s * NC + c

    base = ((w * (TOTAL_STAGES - N_STAGES)) // (NW - 1)) * STAGE_E
    prev = (((w - 1) * (TOTAL_STAGES - N_STAGES)) // (NW - 1)) * STAGE_E
    lo = jnp.where(w == 0, 0, prev + E_PER_TILE)
    hi = jnp.minimum(base + E_PER_TILE, N_EDGES)

    bufs = (ea_buf0, ea_buf1, ea_buf2)
    sems = (sem0, sem1, sem2)
    pend = {}
    for st0 in range(2):
        pend[st0] = pltpu.async_copy(
            ea_hbm.at[pl.ds(base + st0 * STAGE_E, STAGE_E)],
            bufs[st0], sems[st0])
    pltpu.sync_copy(src_hbm.at[pl.ds(base, E_PER_TILE)], raw_buf)

    def fill(i, _):
        ones_buf[i, :] = jnp.full((16,), 1.0, dtype=jnp.float32)
        return 0
    lax.fori_loop(0, CHUNK, fill, 0)

    def zfill(i, _):
        zb[i, :] = jnp.zeros((16,), dtype=jnp.float32)
        return 0
    lax.fori_loop(0, ROWS_PER_TILE, zfill, 0)

    lane = lax.iota(jnp.int32, 16)
    cvec = 128 * (lane % 8) + lane // 8

    def fix(i, _):
        st_off = (i // CHUNKS_PER_STAGE) * STAGE_E
        for g in range(CHUNK // 16):
            ein = cvec + (i % CHUNKS_PER_STAGE) * 16 + 2 * g
            v = plsc.load_gather(raw_buf, [st_off + ein])
            slot = base + st_off + ein
            ok = (slot >= lo) & (slot < hi)
            idx_buf[i, pl.ds(g * 16, 16)] = jnp.where(
                ok, v, jnp.full((16,), DUMP_ROW, dtype=jnp.int32))
        return 0
    lax.fori_loop(0, N_CHUNKS, fix, 0)

    off = s * ROWS_PER_TILE
    pltpu.sync_copy(zb, acc.at[pl.ds(off, ROWS_PER_TILE)])
    pltpu.sync_copy(zb, cnt.at[pl.ds(off, ROWS_PER_TILE)])
    plsc.subcore_barrier()

    scat_pend = {}
    for st in range(N_STAGES):
        if st - 1 in scat_pend:
            for d in scat_pend.pop(st - 1):
                d.wait()
        if st + 2 < N_STAGES:
            pend[st + 2] = pltpu.async_copy(
                ea_hbm.at[pl.ds(base + (st + 2) * STAGE_E, STAGE_E)],
                bufs[(st + 2) % 3], sems[(st + 2) % 3])
        buf = bufs[st % 3]
        pend.pop(st).wait()
        scats = []
        for j in range(CHUNKS_PER_STAGE):
            k = st * CHUNKS_PER_STAGE + j
            scats.append(pltpu.async_copy(
                buf.at[pl.ds(j * CHUNK, CHUNK)], acc.at[idx_buf.at[k]],
                ssem, add=True))
            scats.append(pltpu.async_copy(
                ones_buf, cnt.at[idx_buf.at[k]], ssem, add=True))
        scat_pend[st] = scats
    for st in sorted(scat_pend):
        for d in scat_pend.pop(st):
            d.wait()

    plsc.subcore_barrier()

    pltpu.sync_copy(acc.at[pl.ds(off, ROWS_PER_TILE)], zb)
    pltpu.sync_copy(zb, sums_out.at[c, pl.ds(off, ROWS_PER_TILE)])
    pltpu.sync_copy(cnt.at[pl.ds(off, ROWS_PER_TILE)], zb)
    pltpu.sync_copy(zb, cnts_out.at[c, pl.ds(off, ROWS_PER_TILE)])


def _sc_scatter(src, ea_rows):
    mesh = plsc.VectorSubcoreMesh(core_axis_name="c", subcore_axis_name="s")
    fn = pl.kernel(
        _sc_scatter_body,
        out_type=(
            jax.ShapeDtypeStruct((NC, N_NODES_P, 16), jnp.float32),
            jax.ShapeDtypeStruct((NC, N_NODES_P, 16), jnp.float32),
        ),
        mesh=mesh,
        compiler_params=pltpu.CompilerParams(use_tc_tiling_on_sc=False,
                                             needs_layout_passes=False),
        scratch_types=[
            pltpu.VMEM((E_PER_TILE,), jnp.int32),
            pltpu.VMEM((N_CHUNKS, CHUNK), jnp.int32),
            pltpu.VMEM((STAGE_E, 16), jnp.float32),
            pltpu.VMEM((STAGE_E, 16), jnp.float32),
            pltpu.VMEM((STAGE_E, 16), jnp.float32),
            pltpu.VMEM((CHUNK, 16), jnp.float32),
            pltpu.VMEM((ROWS_PER_TILE, 16), jnp.float32),
            pltpu.VMEM_SHARED((N_NODES_P, 16), jnp.float32),
            pltpu.VMEM_SHARED((N_NODES_P, 16), jnp.float32),
            pltpu.SemaphoreType.DMA,
            pltpu.SemaphoreType.DMA,
            pltpu.SemaphoreType.DMA,
            pltpu.SemaphoreType.DMA,
        ],
    )
    return fn(src, ea_rows)


def _mlp_body(x_ref, s_ref, c_ref, b_ref, u_ref, w1x_ref, w1e_ref, w1u_ref,
              b1_ref, w2_ref, b2_ref, o_ref):
    sums = s_ref[0] + s_ref[1]
    counts = c_ref[0, :, 0:1] + c_ref[1, :, 0:1]
    agg = sums / jnp.maximum(counts, 1.0)

    bvec = b_ref[...]
    gids = lax.broadcasted_iota(jnp.int32, (bvec.shape[0], 64), 1)
    onehot = (bvec == gids).astype(jnp.float32)

    uw = jnp.dot(u_ref[...], w1u_ref[...], preferred_element_type=jnp.float32)
    pre = (jnp.dot(x_ref[...], w1x_ref[...], preferred_element_type=jnp.float32)
           + jnp.dot(agg, w1e_ref[...], preferred_element_type=jnp.float32)
           + jnp.dot(onehot, uw, preferred_element_type=jnp.float32)
           + b1_ref[...])
    h = jnp.maximum(pre, 0.0)
    o_ref[...] = jnp.dot(h, w2_ref[...], preferred_element_type=jnp.float32) + b2_ref[...]


def _mlp(x, sums, cnts, batch2d, u, w1x, w1e, w1u, b1, w2, b2):
    n = x.shape[0]
    blk = 1000
    grid = n // blk
    return pl.pallas_call(
        _mlp_body,
        grid=(grid,),
        in_specs=[
            pl.BlockSpec((blk, 128), lambda i: (i, 0)),
            pl.BlockSpec((NC, blk, 16), lambda i: (0, i, 0)),
            pl.BlockSpec((NC, blk, 16), lambda i: (0, i, 0)),
            pl.BlockSpec((blk, 1), lambda i: (i, 0)),
            pl.BlockSpec((64, 128), lambda i: (0, 0)),
            pl.BlockSpec((128, 128), lambda i: (0, 0)),
            pl.BlockSpec((16, 128), lambda i: (0, 0)),
            pl.BlockSpec((128, 128), lambda i: (0, 0)),
            pl.BlockSpec((1, 128), lambda i: (0, 0)),
            pl.BlockSpec((128, 128), lambda i: (0, 0)),
            pl.BlockSpec((1, 128), lambda i: (0, 0)),
        ],
        out_specs=pl.BlockSpec((blk, 128), lambda i: (i, 0)),
        out_shape=jax.ShapeDtypeStruct((n, 128), jnp.float32),
    )(x, sums, cnts, batch2d, u, w1x, w1e, w1u, b1, w2, b2)


@jax.jit
def kernel(x, edge_index, edge_attr, u, batch, W1, b1, W2, b2):
    ea_p, idx_p = _pack(edge_attr.T, edge_index.astype(jnp.int32))
    ea_rows = ea_p.reshape(E_PAD, 16)
    src = idx_p.reshape(E_PAD)

    sums, cnts = _sc_scatter(src, ea_rows)

    w1x = W1[:128]
    w1e = W1[128:144]
    w1u = W1[144:]
    batch2d = batch.astype(jnp.int32).reshape(-1, 1)
    out = _mlp(x, sums, cnts, batch2d, u,
               w1x, w1e, w1u, b1.reshape(1, -1), W2, b2.reshape(1, -1))
    return out


# speedup vs baseline: 1.0064x; 1.0064x over previous
import jax
import jax.numpy as jnp
from jax import lax
from jax.experimental import pallas as pl
from jax.experimental.pallas import tpu as pltpu
from jax.experimental.pallas import tpu_sc as plsc

N_EDGES = 320000
STAGE_E = 1024
N_BLOCKS = 313
E_PAD = N_BLOCKS * STAGE_E
N_NODES_P = 10240
DUMP_ROW = N_NODES_P - 1
E_PER_TILE = 10240
N_STAGES = E_PER_TILE // STAGE_E
CHUNK = 128
CHUNKS_PER_STAGE = STAGE_E // CHUNK
N_CHUNKS = E_PER_TILE // CHUNK
NC = 2
NS = 16
NW = NC * NS
ROWS_PER_TILE = N_NODES_P // NS
TOTAL_STAGES = E_PAD // STAGE_E


def _pack_body(ea_ref, ei_ref, eap_ref, idxp_ref):
    blk = pl.program_id(0)
    a = ea_ref[...]
    rr = lax.broadcasted_iota(jnp.int32, (128, 128), 0)
    cc = lax.broadcasted_iota(jnp.int32, (128, 128), 1)
    ident = (rr == cc).astype(jnp.float32)
    dn = (((1,), (1,)), ((), ()))
    pieces = [
        lax.dot_general(ident, a[:, j * 128:(j + 1) * 128], dn,
                        preferred_element_type=jnp.float32)
        for j in range(8)
    ]
    eap_ref[...] = jnp.concatenate(pieces, axis=1)
    s = ei_ref[0:1, :].reshape(8, 128)
    r = lax.broadcasted_iota(jnp.int32, (8, 128), 0)
    c = lax.broadcasted_iota(jnp.int32, (8, 128), 1)
    slot = blk * STAGE_E + r * 128 + c
    idxp_ref[...] = jnp.where(slot < N_EDGES, s, DUMP_ROW)


def _pack(ea_t, edge_index):
    return pl.pallas_call(
        _pack_body,
        grid=(N_BLOCKS,),
        in_specs=[
            pl.BlockSpec((16, STAGE_E), lambda i: (0, i)),
            pl.BlockSpec((2, STAGE_E), lambda i: (0, i)),
        ],
        out_specs=[
            pl.BlockSpec((128, 128), lambda i: (i, 0)),
            pl.BlockSpec((8, 128), lambda i: (i, 0)),
        ],
        out_shape=[
            jax.ShapeDtypeStruct((E_PAD // 8, 128), jnp.float32),
            jax.ShapeDtypeStruct((E_PAD // 128, 128), jnp.int32),
        ],
    )(ea_t, edge_index)


def _sc_scatter_body(src_hbm, ea_hbm, sums_out, cnts_out,
                     raw_buf, idx_buf, ea_buf0, ea_buf1, ea_buf2, ones_buf,
                     zb, acc, cnt, sem0, sem1, sem2, ssem):
    c = lax.axis_index("c")
    s = lax.axis_index("s")
    w = s * NC + c

    base = ((w * (TOTAL_STAGES - N_STAGES)) // (NW - 1)) * STAGE_E
    prev = (((w - 1) * (TOTAL_STAGES - N_STAGES)) // (NW - 1)) * STAGE_E
    lo = jnp.where(w == 0, 0, prev + E_PER_TILE)
    hi = jnp.minimum(base + E_PER_TILE, N_EDGES)

    bufs = (ea_buf0, ea_buf1, ea_buf2)
    sems = (sem0, sem1, sem2)
    pend = {}
    for st0 in range(2):
        pend[st0] = pltpu.async_copy(
            ea_hbm.at[pl.ds(base + st0 * STAGE_E, STAGE_E)],
            bufs[st0], sems[st0])
    pltpu.sync_copy(src_hbm.at[pl.ds(base, E_PER_TILE)], raw_buf)

    def fill(i, _):
        ones_buf[i, :] = jnp.full((16,), 1.0, dtype=jnp.float32)
        return 0
    lax.fori_loop(0, CHUNK, fill, 0)

    def zfill(i, _):
        zb[i, :] = jnp.zeros((16,), dtype=jnp.float32)
        return 0
    lax.fori_loop(0, ROWS_PER_TILE, zfill, 0)

    lane = lax.iota(jnp.int32, 16)
    cvec = 128 * (lane % 8) + lane // 8

    def fix(i, _):
        st_off = (i // CHUNKS_PER_STAGE) * STAGE_E
        for g in range(CHUNK // 16):
            ein = cvec + (i % CHUNKS_PER_STAGE) * 16 + 2 * g
            v = plsc.load_gather(raw_buf, [st_off + ein])
            slot = base + st_off + ein
            ok = (slot >= lo) & (slot < hi)
            idx_buf[i, pl.ds(g * 16, 16)] = jnp.where(
                ok, v, jnp.full((16,), DUMP_ROW, dtype=jnp.int32))
        return 0
    lax.fori_loop(0, N_CHUNKS, fix, 0)

    off = s * ROWS_PER_TILE
    pltpu.sync_copy(zb, acc.at[pl.ds(off, ROWS_PER_TILE)])
    pltpu.sync_copy(zb, cnt.at[pl.ds(off, ROWS_PER_TILE)])
    plsc.subcore_barrier()

    scat_pend = {}
    for st in range(N_STAGES):
        if st - 1 in scat_pend:
            for d in scat_pend.pop(st - 1):
                d.wait()
        if st + 2 < N_STAGES:
            pend[st + 2] = pltpu.async_copy(
                ea_hbm.at[pl.ds(base + (st + 2) * STAGE_E, STAGE_E)],
                bufs[(st + 2) % 3], sems[(st + 2) % 3])
        buf = bufs[st % 3]
        pend.pop(st).wait()
        scats = []
        for j in range(CHUNKS_PER_STAGE):
            k = st * CHUNKS_PER_STAGE + j
            scats.append(pltpu.async_copy(
                buf.at[pl.ds(j * CHUNK, CHUNK)], acc.at[idx_buf.at[k]],
                ssem, add=True))
            scats.append(pltpu.async_copy(
                ones_buf, cnt.at[idx_buf.at[k]], ssem, add=True))
        scat_pend[st] = scats
    for st in sorted(scat_pend):
        for d in scat_pend.pop(st):
            d.wait()

    plsc.subcore_barrier()

    pltpu.sync_copy(acc.at[pl.ds(off, ROWS_PER_TILE)], zb)
    pltpu.sync_copy(zb, sums_out.at[c, pl.ds(off, ROWS_PER_TILE)])
    pltpu.sync_copy(cnt.at[pl.ds(off, ROWS_PER_TILE)], zb)
    pltpu.sync_copy(zb, cnts_out.at[c, pl.ds(off, ROWS_PER_TILE)])


def _sc_scatter(src, ea_rows):
    mesh = plsc.VectorSubcoreMesh(core_axis_name="c", subcore_axis_name="s")
    fn = pl.kernel(
        _sc_scatter_body,
        out_type=(
            jax.ShapeDtypeStruct((NC, N_NODES_P, 16), jnp.float32),
            jax.ShapeDtypeStruct((NC, N_NODES_P, 16), jnp.float32),
        ),
        mesh=mesh,
        compiler_params=pltpu.CompilerParams(use_tc_tiling_on_sc=False,
                                             needs_layout_passes=False),
        scratch_types=[
            pltpu.VMEM((E_PER_TILE,), jnp.int32),
            pltpu.VMEM((N_CHUNKS, CHUNK), jnp.int32),
            pltpu.VMEM((STAGE_E, 16), jnp.float32),
            pltpu.VMEM((STAGE_E, 16), jnp.float32),
            pltpu.VMEM((STAGE_E, 16), jnp.float32),
            pltpu.VMEM((CHUNK, 16), jnp.float32),
            pltpu.VMEM((ROWS_PER_TILE, 16), jnp.float32),
            pltpu.VMEM_SHARED((N_NODES_P, 16), jnp.float32),
            pltpu.VMEM_SHARED((N_NODES_P, 16), jnp.float32),
            pltpu.SemaphoreType.DMA,
            pltpu.SemaphoreType.DMA,
            pltpu.SemaphoreType.DMA,
            pltpu.SemaphoreType.DMA,
        ],
    )
    return fn(src, ea_rows)


def _mlp_body(x_ref, s_ref, c_ref, b_ref, u_ref, w1x_ref, w1e_ref, w1u_ref,
              b1_ref, w2_ref, b2_ref, o_ref):
    sums = s_ref[0] + s_ref[1]
    counts = c_ref[0, :, 0:1] + c_ref[1, :, 0:1]
    agg = sums / jnp.maximum(counts, 1.0)

    bvec = b_ref[...]
    gids = lax.broadcasted_iota(jnp.int32, (bvec.shape[0], 64), 1)
    onehot = (bvec == gids).astype(jnp.float32)

    uw = jnp.dot(u_ref[...], w1u_ref[...], preferred_element_type=jnp.float32)
    pre = (jnp.dot(x_ref[...], w1x_ref[...], preferred_element_type=jnp.float32)
           + jnp.dot(agg, w1e_ref[...], preferred_element_type=jnp.float32)
           + jnp.dot(onehot, uw, preferred_element_type=jnp.float32)
           + b1_ref[...])
    h = jnp.maximum(pre, 0.0)
    o_ref[...] = jnp.dot(h, w2_ref[...], preferred_element_type=jnp.float32) + b2_ref[...]


def _mlp(x, sums, cnts, batch2d, u, w1x, w1e, w1u, b1, w2, b2):
    n = x.shape[0]
    blk = 1000
    grid = n // blk
    return pl.pallas_call(
        _mlp_body,
        grid=(grid,),
        in_specs=[
            pl.BlockSpec((blk, 128), lambda i: (i, 0)),
            pl.BlockSpec((NC, blk, 16), lambda i: (0, i, 0)),
            pl.BlockSpec((NC, blk, 16), lambda i: (0, i, 0)),
            pl.BlockSpec((blk, 1), lambda i: (i, 0)),
            pl.BlockSpec((64, 128), lambda i: (0, 0)),
            pl.BlockSpec((128, 128), lambda i: (0, 0)),
            pl.BlockSpec((16, 128), lambda i: (0, 0)),
            pl.BlockSpec((128, 128), lambda i: (0, 0)),
            pl.BlockSpec((1, 128), lambda i: (0, 0)),
            pl.BlockSpec((128, 128), lambda i: (0, 0)),
            pl.BlockSpec((1, 128), lambda i: (0, 0)),
        ],
        out_specs=pl.BlockSpec((blk, 128), lambda i: (i, 0)),
        out_shape=jax.ShapeDtypeStruct((n, 128), jnp.float32),
    )(x, sums, cnts, batch2d, u, w1x, w1e, w1u, b1, w2, b2)


@jax.jit
def kernel(x, edge_index, edge_attr, u, batch, W1, b1, W2, b2):
    ea_p, idx_p = _pack(edge_attr.T, edge_index.astype(jnp.int32))
    ea_rows = ea_p.reshape(E_PAD, 16)
    src = idx_p.reshape(E_PAD)

    sums, cnts = _sc_scatter(src, ea_rows)

    w1x = W1[:128]
    w1e = W1[128:144]
    w1u = W1[144:]
    batch2d = batch.astype(jnp.int32).reshape(-1, 1)
    out = _mlp(x, sums, cnts, batch2d, u,
               w1x, w1e, w1u, b1.reshape(1, -1), W2, b2.reshape(1, -1))
    return out


# speedup vs baseline: 1.9908x; 1.9781x over previous
import jax
import jax.numpy as jnp
from jax import lax
from jax.experimental import pallas as pl
from jax.experimental.pallas import tpu as pltpu
from jax.experimental.pallas import tpu_sc as plsc

N_EDGES = 320000
STAGE_E = 1024
PACK_E = 8192
N_BLOCKS = 40
E_PAD = N_BLOCKS * PACK_E
N_NODES_P = 10240
DUMP_ROW = N_NODES_P - 1
E_PER_TILE = 10240
N_STAGES = E_PER_TILE // STAGE_E
CHUNK = 128
CHUNKS_PER_STAGE = STAGE_E // CHUNK
N_CHUNKS = E_PER_TILE // CHUNK
NC = 2
NS = 16
NW = NC * NS
ROWS_PER_TILE = N_NODES_P // NS


def _pack_body(ea_ref, ei_ref, eap_ref, idxp_ref):
    blk = pl.program_id(0)
    a = ea_ref[...]
    rr = lax.broadcasted_iota(jnp.int32, (128, 128), 0)
    cc = lax.broadcasted_iota(jnp.int32, (128, 128), 1)
    ident = (rr == cc).astype(jnp.float32)
    dn = (((1,), (1,)), ((), ()))
    for s8 in range(PACK_E // STAGE_E):
        pieces = [
            lax.dot_general(
                ident, a[:, s8 * STAGE_E + j * 128:s8 * STAGE_E + (j + 1) * 128],
                dn, preferred_element_type=jnp.float32)
            for j in range(8)
        ]
        eap_ref[s8 * 128:(s8 + 1) * 128, :] = jnp.concatenate(pieces, axis=1)
    s = ei_ref[0:1, :].reshape(PACK_E // 128, 128)
    r = lax.broadcasted_iota(jnp.int32, (PACK_E // 128, 128), 0)
    c = lax.broadcasted_iota(jnp.int32, (PACK_E // 128, 128), 1)
    slot = blk * PACK_E + r * 128 + c
    idxp_ref[...] = jnp.where(slot < N_EDGES, s, DUMP_ROW)


def _pack(ea_t, edge_index):
    return pl.pallas_call(
        _pack_body,
        grid=(N_BLOCKS,),
        in_specs=[
            pl.BlockSpec((16, PACK_E), lambda i: (0, i)),
            pl.BlockSpec((2, PACK_E), lambda i: (0, i)),
        ],
        out_specs=[
            pl.BlockSpec((PACK_E // 8, 128), lambda i: (i, 0)),
            pl.BlockSpec((PACK_E // 128, 128), lambda i: (i, 0)),
        ],
        out_shape=[
            jax.ShapeDtypeStruct((E_PAD // 8, 128), jnp.float32),
            jax.ShapeDtypeStruct((E_PAD // 128, 128), jnp.int32),
        ],
    )(ea_t, edge_index)


def _sc_scatter_body(src_hbm, ea_hbm, sums_out, cnts_out,
                     raw_buf, idx_buf, ea_buf0, ea_buf1, ea_buf2, ones_buf,
                     zb, acc, cnt, sem0, sem1, sem2, ssem):
    c = lax.axis_index("c")
    s = lax.axis_index("s")
    w = s * NC + c

    base = w * E_PER_TILE

    bufs = (ea_buf0, ea_buf1, ea_buf2)
    sems = (sem0, sem1, sem2)
    pend = {}
    for st0 in range(2):
        pend[st0] = pltpu.async_copy(
            ea_hbm.at[pl.ds(base + st0 * STAGE_E, STAGE_E)],
            bufs[st0], sems[st0])
    pltpu.sync_copy(src_hbm.at[pl.ds(base, E_PER_TILE)], raw_buf)

    def fill(i, _):
        ones_buf[i, :] = jnp.full((16,), 1.0, dtype=jnp.float32)
        return 0
    lax.fori_loop(0, CHUNK, fill, 0)

    def zfill(i, _):
        zb[i, :] = jnp.zeros((16,), dtype=jnp.float32)
        return 0
    lax.fori_loop(0, ROWS_PER_TILE, zfill, 0)

    lane = lax.iota(jnp.int32, 16)
    cvec = 128 * (lane % 8) + lane // 8

    def fix(i, _):
        st_off = (i // CHUNKS_PER_STAGE) * STAGE_E
        for g in range(CHUNK // 16):
            ein = cvec + (i % CHUNKS_PER_STAGE) * 16 + 2 * g
            idx_buf[i, pl.ds(g * 16, 16)] = plsc.load_gather(
                raw_buf, [st_off + ein])
        return 0
    lax.fori_loop(0, N_CHUNKS, fix, 0)

    off = s * ROWS_PER_TILE
    pltpu.sync_copy(zb, acc.at[pl.ds(off, ROWS_PER_TILE)])
    pltpu.sync_copy(zb, cnt.at[pl.ds(off, ROWS_PER_TILE)])
    plsc.subcore_barrier()

    scat_pend = {}
    for st in range(N_STAGES):
        if st - 1 in scat_pend:
            for d in scat_pend.pop(st - 1):
                d.wait()
        if st + 2 < N_STAGES:
            pend[st + 2] = pltpu.async_copy(
                ea_hbm.at[pl.ds(base + (st + 2) * STAGE_E, STAGE_E)],
                bufs[(st + 2) % 3], sems[(st + 2) % 3])
        buf = bufs[st % 3]
        pend.pop(st).wait()
        scats = []
        for j in range(CHUNKS_PER_STAGE):
            k = st * CHUNKS_PER_STAGE + j
            scats.append(pltpu.async_copy(
                buf.at[pl.ds(j * CHUNK, CHUNK)], acc.at[idx_buf.at[k]],
                ssem, add=True))
            scats.append(pltpu.async_copy(
                ones_buf, cnt.at[idx_buf.at[k]], ssem, add=True))
        scat_pend[st] = scats
    for st in sorted(scat_pend):
        for d in scat_pend.pop(st):
            d.wait()

    plsc.subcore_barrier()

    pltpu.sync_copy(acc.at[pl.ds(off, ROWS_PER_TILE)], zb)
    pltpu.sync_copy(zb, sums_out.at[c, pl.ds(off, ROWS_PER_TILE)])
    pltpu.sync_copy(cnt.at[pl.ds(off, ROWS_PER_TILE)], zb)
    pltpu.sync_copy(zb, cnts_out.at[c, pl.ds(off, ROWS_PER_TILE)])


def _sc_scatter(src, ea_rows):
    mesh = plsc.VectorSubcoreMesh(core_axis_name="c", subcore_axis_name="s")
    fn = pl.kernel(
        _sc_scatter_body,
        out_type=(
            jax.ShapeDtypeStruct((NC, N_NODES_P, 16), jnp.float32),
            jax.ShapeDtypeStruct((NC, N_NODES_P, 16), jnp.float32),
        ),
        mesh=mesh,
        compiler_params=pltpu.CompilerParams(use_tc_tiling_on_sc=False,
                                             needs_layout_passes=False),
        scratch_types=[
            pltpu.VMEM((E_PER_TILE,), jnp.int32),
            pltpu.VMEM((N_CHUNKS, CHUNK), jnp.int32),
            pltpu.VMEM((STAGE_E, 16), jnp.float32),
            pltpu.VMEM((STAGE_E, 16), jnp.float32),
            pltpu.VMEM((STAGE_E, 16), jnp.float32),
            pltpu.VMEM((CHUNK, 16), jnp.float32),
            pltpu.VMEM((ROWS_PER_TILE, 16), jnp.float32),
            pltpu.VMEM_SHARED((N_NODES_P, 16), jnp.float32),
            pltpu.VMEM_SHARED((N_NODES_P, 16), jnp.float32),
            pltpu.SemaphoreType.DMA,
            pltpu.SemaphoreType.DMA,
            pltpu.SemaphoreType.DMA,
            pltpu.SemaphoreType.DMA,
        ],
    )
    return fn(src, ea_rows)


def _mlp_body(x_ref, s_ref, c_ref, b_ref, u_ref, w1x_ref, w1e_ref, w1u_ref,
              b1_ref, w2_ref, b2_ref, o_ref):
    sums = s_ref[0] + s_ref[1]
    counts = c_ref[0, :, 0:1] + c_ref[1, :, 0:1]
    agg = sums / jnp.maximum(counts, 1.0)

    bvec = b_ref[...]
    gids = lax.broadcasted_iota(jnp.int32, (bvec.shape[0], 64), 1)
    onehot = (bvec == gids).astype(jnp.float32)

    uw = jnp.dot(u_ref[...], w1u_ref[...], preferred_element_type=jnp.float32)
    pre = (jnp.dot(x_ref[...], w1x_ref[...], preferred_element_type=jnp.float32)
           + jnp.dot(agg, w1e_ref[...], preferred_element_type=jnp.float32)
           + jnp.dot(onehot, uw, preferred_element_type=jnp.float32)
           + b1_ref[...])
    h = jnp.maximum(pre, 0.0)
    o_ref[...] = jnp.dot(h, w2_ref[...], preferred_element_type=jnp.float32) + b2_ref[...]


def _mlp(x, sums, cnts, batch2d, u, w1x, w1e, w1u, b1, w2, b2):
    n = x.shape[0]
    blk = 1000
    grid = n // blk
    return pl.pallas_call(
        _mlp_body,
        grid=(grid,),
        in_specs=[
            pl.BlockSpec((blk, 128), lambda i: (i, 0)),
            pl.BlockSpec((NC, blk, 16), lambda i: (0, i, 0)),
            pl.BlockSpec((NC, blk, 16), lambda i: (0, i, 0)),
            pl.BlockSpec((blk, 1), lambda i: (i, 0)),
            pl.BlockSpec((64, 128), lambda i: (0, 0)),
            pl.BlockSpec((128, 128), lambda i: (0, 0)),
            pl.BlockSpec((16, 128), lambda i: (0, 0)),
            pl.BlockSpec((128, 128), lambda i: (0, 0)),
            pl.BlockSpec((1, 128), lambda i: (0, 0)),
            pl.BlockSpec((128, 128), lambda i: (0, 0)),
            pl.BlockSpec((1, 128), lambda i: (0, 0)),
        ],
        out_specs=pl.BlockSpec((blk, 128), lambda i: (i, 0)),
        out_shape=jax.ShapeDtypeStruct((n, 128), jnp.float32),
    )(x, sums, cnts, batch2d, u, w1x, w1e, w1u, b1, w2, b2)


@jax.jit
def kernel(x, edge_index, edge_attr, u, batch, W1, b1, W2, b2):
    ea_p, idx_p = _pack(edge_attr.T, edge_index.astype(jnp.int32))
    ea_rows = ea_p.reshape(E_PAD, 16)
    src = idx_p.reshape(E_PAD)

    sums, cnts = _sc_scatter(src, ea_rows)

    w1x = W1[:128]
    w1e = W1[128:144]
    w1u = W1[144:]
    batch2d = batch.astype(jnp.int32).reshape(-1, 1)
    out = _mlp(x, sums, cnts, batch2d, u,
               w1x, w1e, w1u, b1.reshape(1, -1), W2, b2.reshape(1, -1))
    return out


# speedup vs baseline: 2.2446x; 1.1275x over previous
import jax
import jax.numpy as jnp
from jax import lax
from jax.experimental import pallas as pl
from jax.experimental.pallas import tpu as pltpu
from jax.experimental.pallas import tpu_sc as plsc

N_EDGES = 320000
STAGE_E = 1024
PACK_E = 8192
N_BLOCKS = 40
E_PAD = N_BLOCKS * PACK_E
N_NODES_P = 10240
DUMP_ROW = N_NODES_P - 1
E_PER_TILE = 10240
N_STAGES = E_PER_TILE // STAGE_E
CHUNK = 128
CHUNKS_PER_STAGE = STAGE_E // CHUNK
N_CHUNKS = E_PER_TILE // CHUNK
NC = 2
NS = 16
NW = NC * NS
ROWS_PER_TILE = N_NODES_P // NS


def _pack_body(ea_ref, ei_ref, eap_ref, idxp_ref):
    blk = pl.program_id(0)
    a = ea_ref[...]
    rr = lax.broadcasted_iota(jnp.int32, (128, 128), 0)
    cc = lax.broadcasted_iota(jnp.int32, (128, 128), 1)
    ident = (rr == cc).astype(jnp.float32)
    dn = (((1,), (1,)), ((), ()))
    for s8 in range(PACK_E // STAGE_E):
        pieces = [
            lax.dot_general(
                ident, a[:, s8 * STAGE_E + j * 128:s8 * STAGE_E + (j + 1) * 128],
                dn, preferred_element_type=jnp.float32)
            for j in range(8)
        ]
        eap_ref[s8 * 128:(s8 + 1) * 128, :] = jnp.concatenate(pieces, axis=1)
    s = ei_ref[0:1, :].reshape(PACK_E // 128, 128)
    r = lax.broadcasted_iota(jnp.int32, (PACK_E // 128, 128), 0)
    c = lax.broadcasted_iota(jnp.int32, (PACK_E // 128, 128), 1)
    slot = blk * PACK_E + r * 128 + c
    idxp_ref[...] = jnp.where(slot < N_EDGES, s, DUMP_ROW)


def _pack(ea_t, edge_index):
    return pl.pallas_call(
        _pack_body,
        grid=(N_BLOCKS,),
        in_specs=[
            pl.BlockSpec((16, PACK_E), lambda i: (0, i)),
            pl.BlockSpec((2, PACK_E), lambda i: (0, i)),
        ],
        out_specs=[
            pl.BlockSpec((PACK_E // 8, 128), lambda i: (i, 0)),
            pl.BlockSpec((PACK_E // 128, 128), lambda i: (i, 0)),
        ],
        out_shape=[
            jax.ShapeDtypeStruct((E_PAD // 8, 128), jnp.float32),
            jax.ShapeDtypeStruct((E_PAD // 128, 128), jnp.int32),
        ],
    )(ea_t, edge_index)


def _sc_scatter_body(src_hbm, ea_hbm, sums_out, cnts_out,
                     raw_buf, idx_buf, ea_buf0, ea_buf1, ea_buf2, ones_buf,
                     zb, zb128, acc, cnt, sem0, sem1, sem2, ssem):
    c = lax.axis_index("c")
    s = lax.axis_index("s")
    w = s * NC + c

    base = w * E_PER_TILE

    bufs = (ea_buf0, ea_buf1, ea_buf2)
    sems = (sem0, sem1, sem2)
    pend = {}
    for st0 in range(2):
        pend[st0] = pltpu.async_copy(
            ea_hbm.at[pl.ds(base + st0 * STAGE_E, STAGE_E)],
            bufs[st0], sems[st0])
    pltpu.sync_copy(src_hbm.at[pl.ds(base, E_PER_TILE)], raw_buf)

    def fill(i, _):
        ones_buf[i, :] = jnp.full((16,), 1.0, dtype=jnp.float32)
        return 0
    lax.fori_loop(0, CHUNK, fill, 0)

    def zfill(i, _):
        zb[i, :] = jnp.zeros((16,), dtype=jnp.float32)
        return 0
    lax.fori_loop(0, ROWS_PER_TILE, zfill, 0)

    lane = lax.iota(jnp.int32, 16)
    cvec = 128 * (lane % 8) + lane // 8

    def fix(i, _):
        st_off = (i // CHUNKS_PER_STAGE) * STAGE_E
        for g in range(CHUNK // 16):
            ein = cvec + (i % CHUNKS_PER_STAGE) * 16 + 2 * g
            idx_buf[i, pl.ds(g * 16, 16)] = plsc.load_gather(
                raw_buf, [st_off + ein])
        return 0
    lax.fori_loop(0, N_CHUNKS, fix, 0)

    off = s * ROWS_PER_TILE
    pltpu.sync_copy(zb, acc.at[pl.ds(off, ROWS_PER_TILE)])
    pltpu.sync_copy(zb, cnt.at[pl.ds(off, ROWS_PER_TILE)])
    plsc.subcore_barrier()

    scat_pend = {}
    for st in range(N_STAGES):
        if st - 1 in scat_pend:
            for d in scat_pend.pop(st - 1):
                d.wait()
        if st + 2 < N_STAGES:
            pend[st + 2] = pltpu.async_copy(
                ea_hbm.at[pl.ds(base + (st + 2) * STAGE_E, STAGE_E)],
                bufs[(st + 2) % 3], sems[(st + 2) % 3])
        buf = bufs[st % 3]
        pend.pop(st).wait()
        scats = []
        for j in range(CHUNKS_PER_STAGE):
            k = st * CHUNKS_PER_STAGE + j
            scats.append(pltpu.async_copy(
                buf.at[pl.ds(j * CHUNK, CHUNK)], acc.at[idx_buf.at[k]],
                ssem, add=True))
            scats.append(pltpu.async_copy(
                ones_buf, cnt.at[idx_buf.at[k]], ssem, add=True))
        scat_pend[st] = scats
    for st in sorted(scat_pend):
        for d in scat_pend.pop(st):
            d.wait()

    plsc.subcore_barrier()

    poff = s * (ROWS_PER_TILE // 8)

    def repack(r, _):
        for j in range(8):
            zb128[r, pl.ds(j * 16, 16)] = zb[r * 8 + j, :]
        return 0

    pltpu.sync_copy(acc.at[pl.ds(off, ROWS_PER_TILE)], zb)
    lax.fori_loop(0, ROWS_PER_TILE // 8, repack, 0)
    pltpu.sync_copy(zb128, sums_out.at[c, pl.ds(poff, ROWS_PER_TILE // 8)])
    pltpu.sync_copy(cnt.at[pl.ds(off, ROWS_PER_TILE)], zb)
    lax.fori_loop(0, ROWS_PER_TILE // 8, repack, 0)
    pltpu.sync_copy(zb128, cnts_out.at[c, pl.ds(poff, ROWS_PER_TILE // 8)])


def _sc_scatter(src, ea_rows):
    mesh = plsc.VectorSubcoreMesh(core_axis_name="c", subcore_axis_name="s")
    fn = pl.kernel(
        _sc_scatter_body,
        out_type=(
            jax.ShapeDtypeStruct((NC, N_NODES_P // 8, 128), jnp.float32),
            jax.ShapeDtypeStruct((NC, N_NODES_P // 8, 128), jnp.float32),
        ),
        mesh=mesh,
        compiler_params=pltpu.CompilerParams(use_tc_tiling_on_sc=False,
                                             needs_layout_passes=False),
        scratch_types=[
            pltpu.VMEM((E_PER_TILE,), jnp.int32),
            pltpu.VMEM((N_CHUNKS, CHUNK), jnp.int32),
            pltpu.VMEM((STAGE_E, 16), jnp.float32),
            pltpu.VMEM((STAGE_E, 16), jnp.float32),
            pltpu.VMEM((STAGE_E, 16), jnp.float32),
            pltpu.VMEM((CHUNK, 16), jnp.float32),
            pltpu.VMEM((ROWS_PER_TILE, 16), jnp.float32),
            pltpu.VMEM((ROWS_PER_TILE // 8, 128), jnp.float32),
            pltpu.VMEM_SHARED((N_NODES_P, 16), jnp.float32),
            pltpu.VMEM_SHARED((N_NODES_P, 16), jnp.float32),
            pltpu.SemaphoreType.DMA,
            pltpu.SemaphoreType.DMA,
            pltpu.SemaphoreType.DMA,
            pltpu.SemaphoreType.DMA,
        ],
    )
    return fn(src, ea_rows)


def _mlp_body(x_ref, s_ref, c_ref, b_ref, u_ref, w1x_ref, w1eb_ref, w1u_ref,
              b1_ref, w2_ref, b2_ref, o_ref):
    blk = x_ref.shape[0]
    sums_p = s_ref[0] + s_ref[1]
    cnts_p = c_ref[0] + c_ref[1]
    agg_p = sums_p / jnp.maximum(cnts_p, 1.0)
    agg_c = jnp.dot(agg_p, w1eb_ref[...],
                    preferred_element_type=jnp.float32)
    agg_c = agg_c.reshape(blk, 128)

    bvec = b_ref[...]
    gids = lax.broadcasted_iota(jnp.int32, (blk, 64), 1)
    onehot = (bvec == gids).astype(jnp.float32)

    uw = jnp.dot(u_ref[...], w1u_ref[...], preferred_element_type=jnp.float32)
    pre = (jnp.dot(x_ref[...], w1x_ref[...], preferred_element_type=jnp.float32)
           + agg_c
           + jnp.dot(onehot, uw, preferred_element_type=jnp.float32)
           + b1_ref[...])
    h = jnp.maximum(pre, 0.0)
    o_ref[...] = jnp.dot(h, w2_ref[...], preferred_element_type=jnp.float32) + b2_ref[...]


def _mlp(x, sums, cnts, batch2d, u, w1x, w1e_big, w1u, b1, w2, b2):
    n = x.shape[0]
    blk = 1024
    grid = (n + blk - 1) // blk
    return pl.pallas_call(
        _mlp_body,
        grid=(grid,),
        in_specs=[
            pl.BlockSpec((blk, 128), lambda i: (i, 0)),
            pl.BlockSpec((NC, blk // 8, 128), lambda i: (0, i, 0)),
            pl.BlockSpec((NC, blk // 8, 128), lambda i: (0, i, 0)),
            pl.BlockSpec((blk, 1), lambda i: (i, 0)),
            pl.BlockSpec((64, 128), lambda i: (0, 0)),
            pl.BlockSpec((128, 128), lambda i: (0, 0)),
            pl.BlockSpec((128, 1024), lambda i: (0, 0)),
            pl.BlockSpec((128, 128), lambda i: (0, 0)),
            pl.BlockSpec((1, 128), lambda i: (0, 0)),
            pl.BlockSpec((128, 128), lambda i: (0, 0)),
            pl.BlockSpec((1, 128), lambda i: (0, 0)),
        ],
        out_specs=pl.BlockSpec((blk, 128), lambda i: (i, 0)),
        out_shape=jax.ShapeDtypeStruct((n, 128), jnp.float32),
    )(x, sums, cnts, batch2d, u, w1x, w1e_big, w1u, b1, w2, b2)


@jax.jit
def kernel(x, edge_index, edge_attr, u, batch, W1, b1, W2, b2):
    ea_p, idx_p = _pack(edge_attr.T, edge_index.astype(jnp.int32))
    ea_rows = ea_p.reshape(E_PAD, 16)
    src = idx_p.reshape(E_PAD)

    sums, cnts = _sc_scatter(src, ea_rows)

    w1x = W1[:128]
    w1e_big = jnp.kron(jnp.eye(8, dtype=W1.dtype), W1[128:144])
    w1u = W1[144:]
    batch2d = batch.astype(jnp.int32).reshape(-1, 1)
    out = _mlp(x, sums, cnts, batch2d, u,
               w1x, w1e_big, w1u, b1.reshape(1, -1), W2, b2.reshape(1, -1))
    return out


# speedup vs baseline: 2.6389x; 1.1757x over previous
import jax
import jax.numpy as jnp
from jax import lax
from jax.experimental import pallas as pl
from jax.experimental.pallas import tpu as pltpu
from jax.experimental.pallas import tpu_sc as plsc

N_EDGES = 320000
STAGE_E = 1024
PACK_E = 8192
N_BLOCKS = 40
E_PAD = N_BLOCKS * PACK_E
N_NODES_P = 10240
DUMP_ROW = N_NODES_P - 1
E_PER_TILE = 10240
N_STAGES = E_PER_TILE // STAGE_E
CHUNK = 128
CHUNKS_PER_STAGE = STAGE_E // CHUNK
N_CHUNKS = E_PER_TILE // CHUNK
NC = 2
NS = 16
NW = NC * NS
ROWS_PER_TILE = N_NODES_P // NS


def _pack_body(ea_ref, ei_ref, eap_ref, idxp_ref):
    blk = pl.program_id(0)
    a = ea_ref[...]
    rr = lax.broadcasted_iota(jnp.int32, (128, 128), 0)
    cc = lax.broadcasted_iota(jnp.int32, (128, 128), 1)
    ident = (rr == cc).astype(jnp.float32)
    dn = (((1,), (1,)), ((), ()))
    for s8 in range(PACK_E // STAGE_E):
        stacked = jnp.concatenate(
            [a[:, s8 * STAGE_E + j * 128:s8 * STAGE_E + (j + 1) * 128]
             for j in range(8)], axis=0)
        eap_ref[s8 * 128:(s8 + 1) * 128, :] = lax.dot_general(
            ident, stacked, dn, preferred_element_type=jnp.float32)
    s = ei_ref[0:1, :].reshape(PACK_E // 128, 128)
    r = lax.broadcasted_iota(jnp.int32, (PACK_E // 128, 128), 0)
    c = lax.broadcasted_iota(jnp.int32, (PACK_E // 128, 128), 1)
    slot = blk * PACK_E + r * 128 + c
    idxp_ref[...] = jnp.where(slot < N_EDGES, s, DUMP_ROW)


def _pack(ea_t, edge_index):
    return pl.pallas_call(
        _pack_body,
        grid=(N_BLOCKS,),
        in_specs=[
            pl.BlockSpec((16, PACK_E), lambda i: (0, i)),
            pl.BlockSpec((2, PACK_E), lambda i: (0, i)),
        ],
        out_specs=[
            pl.BlockSpec((PACK_E // 8, 128), lambda i: (i, 0)),
            pl.BlockSpec((PACK_E // 128, 128), lambda i: (i, 0)),
        ],
        out_shape=[
            jax.ShapeDtypeStruct((E_PAD // 8, 128), jnp.float32),
            jax.ShapeDtypeStruct((E_PAD // 128, 128), jnp.int32),
        ],
    )(ea_t, edge_index)


def _sc_scatter_body(src_hbm, ea_hbm, sums_out, cnts_out,
                     raw_buf, idx_buf, ea_buf0, ea_buf1, ea_buf2, ones_buf,
                     zb, zb128, acc, cnt, sem0, sem1, sem2, ssem):
    c = lax.axis_index("c")
    s = lax.axis_index("s")
    w = s * NC + c

    base = w * E_PER_TILE

    bufs = (ea_buf0, ea_buf1, ea_buf2)
    sems = (sem0, sem1, sem2)
    pend = {}
    for st0 in range(2):
        pend[st0] = pltpu.async_copy(
            ea_hbm.at[pl.ds(base + st0 * STAGE_E, STAGE_E)],
            bufs[st0], sems[st0])
    pltpu.sync_copy(src_hbm.at[pl.ds(base, E_PER_TILE)], raw_buf)

    def fill(i, _):
        ones_buf[i, :] = jnp.full((16,), 1.0, dtype=jnp.float32)
        return 0
    lax.fori_loop(0, CHUNK, fill, 0)

    def zfill(i, _):
        zb[i, :] = jnp.zeros((16,), dtype=jnp.float32)
        return 0
    lax.fori_loop(0, ROWS_PER_TILE, zfill, 0)

    lane = lax.iota(jnp.int32, 16)
    cvec = 128 * (lane % 8) + lane // 8

    def fix(i, _):
        st_off = (i // CHUNKS_PER_STAGE) * STAGE_E
        for g in range(CHUNK // 16):
            ein = cvec + (i % CHUNKS_PER_STAGE) * 16 + 2 * g
            idx_buf[i, pl.ds(g * 16, 16)] = plsc.load_gather(
                raw_buf, [st_off + ein])
        return 0
    lax.fori_loop(0, N_CHUNKS, fix, 0)

    off = s * ROWS_PER_TILE
    pltpu.sync_copy(zb, acc.at[pl.ds(off, ROWS_PER_TILE)])
    pltpu.sync_copy(zb, cnt.at[pl.ds(off, ROWS_PER_TILE)])
    plsc.subcore_barrier()

    scat_pend = {}
    for st in range(N_STAGES):
        if st - 1 in scat_pend:
            for d in scat_pend.pop(st - 1):
                d.wait()
        if st + 2 < N_STAGES:
            pend[st + 2] = pltpu.async_copy(
                ea_hbm.at[pl.ds(base + (st + 2) * STAGE_E, STAGE_E)],
                bufs[(st + 2) % 3], sems[(st + 2) % 3])
        buf = bufs[st % 3]
        pend.pop(st).wait()
        scats = []
        for j in range(CHUNKS_PER_STAGE):
            k = st * CHUNKS_PER_STAGE + j
            scats.append(pltpu.async_copy(
                buf.at[pl.ds(j * CHUNK, CHUNK)], acc.at[idx_buf.at[k]],
                ssem, add=True))
            scats.append(pltpu.async_copy(
                ones_buf, cnt.at[idx_buf.at[k]], ssem, add=True))
        scat_pend[st] = scats
    for st in sorted(scat_pend):
        for d in scat_pend.pop(st):
            d.wait()

    plsc.subcore_barrier()

    poff = s * (ROWS_PER_TILE // 8)

    def repack(r, _):
        for j in range(8):
            zb128[r, pl.ds(j * 16, 16)] = zb[r * 8 + j, :]
        return 0

    pltpu.sync_copy(acc.at[pl.ds(off, ROWS_PER_TILE)], zb)
    lax.fori_loop(0, ROWS_PER_TILE // 8, repack, 0)
    pltpu.sync_copy(zb128, sums_out.at[c, pl.ds(poff, ROWS_PER_TILE // 8)])
    pltpu.sync_copy(cnt.at[pl.ds(off, ROWS_PER_TILE)], zb)
    lax.fori_loop(0, ROWS_PER_TILE // 8, repack, 0)
    pltpu.sync_copy(zb128, cnts_out.at[c, pl.ds(poff, ROWS_PER_TILE // 8)])


def _sc_scatter(src, ea_rows):
    mesh = plsc.VectorSubcoreMesh(core_axis_name="c", subcore_axis_name="s")
    fn = pl.kernel(
        _sc_scatter_body,
        out_type=(
            jax.ShapeDtypeStruct((NC, N_NODES_P // 8, 128), jnp.float32),
            jax.ShapeDtypeStruct((NC, N_NODES_P // 8, 128), jnp.float32),
        ),
        mesh=mesh,
        compiler_params=pltpu.CompilerParams(use_tc_tiling_on_sc=False,
                                             needs_layout_passes=False),
        scratch_types=[
            pltpu.VMEM((E_PER_TILE,), jnp.int32),
            pltpu.VMEM((N_CHUNKS, CHUNK), jnp.int32),
            pltpu.VMEM((STAGE_E, 16), jnp.float32),
            pltpu.VMEM((STAGE_E, 16), jnp.float32),
            pltpu.VMEM((STAGE_E, 16), jnp.float32),
            pltpu.VMEM((CHUNK, 16), jnp.float32),
            pltpu.VMEM((ROWS_PER_TILE, 16), jnp.float32),
            pltpu.VMEM((ROWS_PER_TILE // 8, 128), jnp.float32),
            pltpu.VMEM_SHARED((N_NODES_P, 16), jnp.float32),
            pltpu.VMEM_SHARED((N_NODES_P, 16), jnp.float32),
            pltpu.SemaphoreType.DMA,
            pltpu.SemaphoreType.DMA,
            pltpu.SemaphoreType.DMA,
            pltpu.SemaphoreType.DMA,
        ],
    )
    return fn(src, ea_rows)


def _mlp_body(x_ref, s_ref, c_ref, b_ref, u_ref, w1x_ref, w1eb_ref, w1u_ref,
              b1_ref, w2_ref, b2_ref, o_ref):
    blk = x_ref.shape[0]
    sums_p = s_ref[0] + s_ref[1]
    cnts_p = c_ref[0] + c_ref[1]
    agg_p = sums_p / jnp.maximum(cnts_p, 1.0)
    agg_c = jnp.dot(agg_p, w1eb_ref[...],
                    preferred_element_type=jnp.float32)
    agg_c = agg_c.reshape(blk, 128)

    bvec = b_ref[...]
    gids = lax.broadcasted_iota(jnp.int32, (blk, 64), 1)
    onehot = (bvec == gids).astype(jnp.float32)

    uw = jnp.dot(u_ref[...], w1u_ref[...], preferred_element_type=jnp.float32)
    pre = (jnp.dot(x_ref[...], w1x_ref[...], preferred_element_type=jnp.float32)
           + agg_c
           + jnp.dot(onehot, uw, preferred_element_type=jnp.float32)
           + b1_ref[...])
    h = jnp.maximum(pre, 0.0)
    o_ref[...] = jnp.dot(h, w2_ref[...], preferred_element_type=jnp.float32) + b2_ref[...]


def _mlp(x, sums, cnts, batch2d, u, w1x, w1e_big, w1u, b1, w2, b2):
    n = x.shape[0]
    blk = 1024
    grid = (n + blk - 1) // blk
    return pl.pallas_call(
        _mlp_body,
        grid=(grid,),
        in_specs=[
            pl.BlockSpec((blk, 128), lambda i: (i, 0)),
            pl.BlockSpec((NC, blk // 8, 128), lambda i: (0, i, 0)),
            pl.BlockSpec((NC, blk // 8, 128), lambda i: (0, i, 0)),
            pl.BlockSpec((blk, 1), lambda i: (i, 0)),
            pl.BlockSpec((64, 128), lambda i: (0, 0)),
            pl.BlockSpec((128, 128), lambda i: (0, 0)),
            pl.BlockSpec((128, 1024), lambda i: (0, 0)),
            pl.BlockSpec((128, 128), lambda i: (0, 0)),
            pl.BlockSpec((1, 128), lambda i: (0, 0)),
            pl.BlockSpec((128, 128), lambda i: (0, 0)),
            pl.BlockSpec((1, 128), lambda i: (0, 0)),
        ],
        out_specs=pl.BlockSpec((blk, 128), lambda i: (i, 0)),
        out_shape=jax.ShapeDtypeStruct((n, 128), jnp.float32),
    )(x, sums, cnts, batch2d, u, w1x, w1e_big, w1u, b1, w2, b2)


@jax.jit
def kernel(x, edge_index, edge_attr, u, batch, W1, b1, W2, b2):
    ea_p, idx_p = _pack(edge_attr.T, edge_index.astype(jnp.int32))
    ea_rows = ea_p.reshape(E_PAD, 16)
    src = idx_p.reshape(E_PAD)

    sums, cnts = _sc_scatter(src, ea_rows)

    w1x = W1[:128]
    w1e_big = jnp.kron(jnp.eye(8, dtype=W1.dtype), W1[128:144])
    w1u = W1[144:]
    batch2d = batch.astype(jnp.int32).reshape(-1, 1)
    out = _mlp(x, sums, cnts, batch2d, u,
               w1x, w1e_big, w1u, b1.reshape(1, -1), W2, b2.reshape(1, -1))
    return out
